# Initial kernel scaffold; baseline (speedup 1.0000x reference)
import jax
import jax.numpy as jnp
from jax.experimental import pallas as pl


def kernel(x, edge_index, candidate_features, W1, b1, W2, b2, W3, b3, Ws1, bs1, Ws2, bs2, We1, be1, We2, be2):
    raise NotImplementedError("write your pallas kernel here")


# speedup vs baseline: 25.0575x; 25.0575x over previous
import functools

import jax
import jax.numpy as jnp
from jax import lax
from jax.experimental import pallas as pl
from jax.experimental.pallas import tpu as pltpu
from jax.experimental.pallas import tpu_sc as plsc

NC = 2
NS = 16
NW = NC * NS
CH = 128
MACRO = 16
N_EDGES = 1600000
EPW = 51200
EPAD = NW * EPW
N_TOT = 100007
DUMMY = 100000
NR = 100096
RPT = NR // NS
W8 = 8


_mesh = plsc.VectorSubcoreMesh(core_axis_name="c", subcore_axis_name="s",
                               num_cores=NC, num_subcores=NS)


@functools.partial(
    pl.kernel,
    out_type=jax.ShapeDtypeStruct((NC, NR, W8), jnp.float32),
    mesh=_mesh,
    scratch_types=[
        pltpu.VMEM((MACRO, CH), jnp.int32),
        pltpu.VMEM((MACRO, CH), jnp.int32),
        pltpu.VMEM((MACRO, CH, W8), jnp.float32),
        pltpu.VMEM_SHARED((NR, W8), jnp.float32),
        pltpu.SemaphoreType.DMA,
    ],
    compiler_params=pltpu.CompilerParams(use_tc_tiling_on_sc=False),
)
def _spmv_sc(src_hbm, dst_hbm, v_hbm, zero_hbm, out_hbm,
             sidx, didx, rows, accum, sem):
    c = lax.axis_index("c")
    s = lax.axis_index("s")
    wid = c * NS + s

    pltpu.sync_copy(zero_hbm, accum.at[pl.ds(s * RPT, RPT)])
    plsc.subcore_barrier()

    row_base = wid * (EPW // CH)

    def macro_body(m, carry):
        r0 = row_base + m * MACRO
        pltpu.sync_copy(src_hbm.at[pl.ds(r0, MACRO)], sidx)
        pltpu.sync_copy(dst_hbm.at[pl.ds(r0, MACRO)], didx)
        descs = []
        for i in range(MACRO):
            descs.append(
                pltpu.async_copy(v_hbm.at[sidx.at[i]], rows.at[i], sem))
        for d in descs:
            d.wait()
        for i in range(MACRO):
            pltpu.sync_copy(rows.at[i], accum.at[didx.at[i]], add=True)
        return carry

    lax.fori_loop(0, EPW // (CH * MACRO), macro_body, 0)

    plsc.subcore_barrier()
    pltpu.sync_copy(accum.at[pl.ds(s * RPT, RPT)],
                    out_hbm.at[c, pl.ds(s * RPT, RPT)])


def _adj_apply(src2d, dst2d, zero, V):
    out = _spmv_sc(src2d, dst2d, V, zero)
    S = out[0] + out[1]
    return jnp.concatenate(
        [S[:DUMMY], jnp.zeros((N_TOT - DUMMY, W8), jnp.float32)], axis=0)


def _msoftmax(v):
    mask = v != 0
    neg = jnp.where(mask, v, -jnp.inf)
    m = jnp.max(neg)
    e = jnp.where(mask, jnp.exp(neg - m), 0.0)
    return e / jnp.sum(e)


def _slog(p):
    return jnp.where(p > 0, jnp.log(jnp.maximum(p, 1e-38)), -jnp.inf)


def kernel(x, edge_index, candidate_features, W1, b1, W2, b2, W3, b3,
           Ws1, bs1, Ws2, bs2, We1, be1, We2, be2):
    n_graph = x.shape[0]
    h0 = jnp.concatenate([x, candidate_features], axis=0).astype(jnp.float32)
    n_tot = h0.shape[0]

    src = edge_index[0]
    dst = edge_index[1]
    pad = jnp.full((EPAD - N_EDGES,), DUMMY, jnp.int32)
    src2d = jnp.concatenate([src, pad]).reshape(EPAD // CH, CH)
    dst2d = jnp.concatenate([dst, pad]).reshape(EPAD // CH, CH)
    zero = jnp.zeros((RPT, W8), jnp.float32)

    ones8 = jnp.ones((n_tot, W8), jnp.float32)
    deg = _adj_apply(src2d, dst2d, zero, ones8)[:, 0] + 1.0
    dinv = jnp.where(deg > 0, lax.rsqrt(deg), 0.0)[:, None]

    u = jnp.concatenate([h0, jnp.ones((n_tot, 1), jnp.float32)], axis=1)

    def ahat(Min):
        S = _adj_apply(src2d, dst2d, zero, Min * dinv)
        return dinv * S + dinv * dinv * Min

    M1 = ahat(u)
    M2 = ahat(M1)
    M3 = ahat(M2)

    C = jnp.concatenate([W1 @ W2 @ W3, (b1 @ W2 @ W3)[None], (b2 @ W3)[None]],
                        axis=0)
    Z = jnp.concatenate([M3[:, :7], M2[:, 7:8], M1[:, 7:8]], axis=1)
    h3 = Z @ C + b3

    s = jnp.clip(h3 @ Ws1 + bs1, 0.0, 6.0) @ Ws2 + bs2
    start_probs = jax.nn.softmax(s, axis=0)
    cand_mask = jnp.ones((n_tot, 1), jnp.float32).at[
        jnp.arange(n_graph, n_tot)].set(0.0)
    start_probs = (start_probs * cand_mask).squeeze(-1)
    start_probs = _msoftmax(start_probs)
    start_node = jax.random.categorical(
        jax.random.key(42), lax.stop_gradient(_slog(start_probs)))
    start_oh = jnp.zeros_like(start_probs).at[start_node].set(1.0)

    e = jnp.clip(h3 @ We1 + be1, 0.0, 6.0) @ We2 + be2
    end_probs = jax.nn.softmax(e, axis=0)
    start_mask = jnp.ones((n_tot, 1), jnp.float32).at[start_node].set(0.0)
    end_probs = (end_probs * start_mask).squeeze(-1)
    end_probs = _msoftmax(end_probs)
    end_node = jax.random.categorical(
        jax.random.key(43), lax.stop_gradient(_slog(end_probs)))
    end_oh = jnp.zeros_like(end_probs).at[end_node].set(1.0)

    dst_new = jnp.where(end_node >= n_graph, n_graph, end_node).astype(
        edge_index.dtype)
    new_edge = jnp.stack([start_node.astype(edge_index.dtype), dst_new])[:, None]
    new_edge_index = jnp.concatenate([edge_index, new_edge], axis=1)
    return (start_probs, start_oh, end_probs, end_oh, new_edge_index)
